# baseline (speedup 1.0000x reference)
import functools

import jax
import jax.numpy as jnp
from jax import lax
from jax.experimental import pallas as pl
from jax.experimental.pallas import tpu as pltpu
from jax.experimental.pallas import tpu_sc as plsc

_f32 = jnp.float32
_i32 = jnp.int32

_NC = 2
_NS = 16
_NW = _NC * _NS
_L = 16

_C = 128
_SMW = 8


def _matmul_call(x, w, bp):
    m, k = x.shape
    _, h = w.shape

    def body(x_ref, w_ref, o_ref):
        o_ref[...] = lax.dot_general(
            x_ref[...], w_ref[...], (((1,), (0,)), ((), ())),
            preferred_element_type=_f32)

    return pl.pallas_call(
        body,
        grid=(m // bp,),
        in_specs=[
            pl.BlockSpec((bp, k), lambda i: (i, 0)),
            pl.BlockSpec((k, h), lambda i: (0, 0)),
        ],
        out_specs=pl.BlockSpec((bp, h), lambda i: (i, 0)),
        out_shape=jax.ShapeDtypeStruct((m, h), _f32),
    )(x, w)


def _matmul_add_call(base, x, w, bp):
    m, k = x.shape
    _, h = w.shape

    def body(b_ref, x_ref, w_ref, o_ref):
        o_ref[...] = b_ref[...] + lax.dot_general(
            x_ref[...], w_ref[...], (((1,), (0,)), ((), ())),
            preferred_element_type=_f32)

    return pl.pallas_call(
        body,
        grid=(m // bp,),
        in_specs=[
            pl.BlockSpec((bp, h), lambda i: (i, 0)),
            pl.BlockSpec((bp, k), lambda i: (i, 0)),
            pl.BlockSpec((k, h), lambda i: (0, 0)),
        ],
        out_specs=pl.BlockSpec((bp, h), lambda i: (i, 0)),
        out_shape=jax.ShapeDtypeStruct((m, h), _f32),
    )(base, x, w)


def _coalesce_call(t0, t1, t2, s0, d0, s1, d1, s2, d2,
                   melem_src, melem_tab, segmeta, msmall, zin,
                   t_total, h):
    sizes = (s0.shape[0], s1.shape[0], s2.shape[0])

    mesh = plsc.VectorSubcoreMesh(
        core_axis_name="c", subcore_axis_name="s",
        num_cores=_NC, num_subcores=_NS)

    @functools.partial(
        pl.kernel,
        out_type=jax.ShapeDtypeStruct((t_total, h), _f32),
        mesh=mesh,
        scratch_types=[
            pltpu.VMEM((_C,), _i32),
            pltpu.VMEM((_C,), _i32),
            pltpu.VMEM((_C, h), _f32),
            pltpu.VMEM((_C, h), _f32),
            pltpu.VMEM((_C,), _i32),
            pltpu.VMEM((_L,), _i32),
            pltpu.VMEM((_L,), _i32),
            pltpu.VMEM((_L,), _i32),
            pltpu.VMEM((_L,), _i32),
            pltpu.VMEM((_L, h), _f32),
            pltpu.VMEM((_L, h), _f32),
            pltpu.VMEM((_L, h), _f32),
            pltpu.VMEM((h,), _f32),
            pltpu.VMEM((_L,), _i32),
            pltpu.SemaphoreType.DMA,
        ],
    )
    def k(t0h, t1h, t2h, s0h, d0h, s1h, d1h, s2h, d2h,
          msrch, mtabh, segh, msmallh, zinh, outh,
          pidx, destv, rows, zrows, ztail, mb16, sidx, stab, gidx,
          rowsA, rowsB, rowsC, accrow, sdst, sem):
        cid = lax.axis_index("c")
        sid = lax.axis_index("s")
        w = cid * _NS + sid
        iota16 = lax.broadcasted_iota(_i32, (_L,), 0)

        pltpu.sync_copy(msmallh, mb16)
        mv = mb16[...]
        n_seg = mv[0]
        n_uniq = mv[1]

        for t, (tab, srch, dsth, sz) in enumerate(
                ((t0h, s0h, d0h, sizes[0]), (t1h, s1h, d1h, sizes[1]),
                 (t2h, s2h, d2h, sizes[2]))):
            nchunks = mv[2 + t]

            def a_body(i, c, tab=tab, srch=srch, dsth=dsth):
                g = w + i * _NW
                off = pl.multiple_of(g * _C, _C)
                pltpu.sync_copy(srch.at[pl.ds(off, _C)], pidx)
                pltpu.sync_copy(dsth.at[pl.ds(off, _C)], destv)
                pltpu.async_copy(tab.at[pidx], rows, sem).wait()
                pltpu.sync_copy(rows, outh.at[destv])
                return c

            trips = jnp.maximum(0, (nchunks - w + _NW - 1) // _NW)
            lax.fori_loop(0, trips, a_body, 0)


        def seg_body(i, c):
            m = w + i * _NW
            soff = pl.multiple_of(m * _SMW, _SMW)
            pltpu.sync_copy(segh.at[pl.ds(soff, _L)], mb16)
            sv = mb16[...]
            sstart = sv[0]
            slen = sv[1]
            srank = sv[2]
            a0 = (sstart // _L) * _L
            lead = sstart - a0
            for q in range(h // _L):
                accrow[pl.ds(q * _L, _L)] = jnp.zeros((_L,), _f32)

            def ch_body(kk, cc):
                eoff = pl.multiple_of(a0, _L) + kk * _L
                pltpu.sync_copy(msrch.at[pl.ds(eoff, _L)], sidx)
                pltpu.sync_copy(mtabh.at[pl.ds(eoff, _L)], stab)
                sv16 = sidx[...]
                tv16 = stab[...]
                gidx[...] = jnp.where(tv16 == 0, sv16, 0)
                pltpu.async_copy(t0h.at[gidx], rowsA, sem).wait()
                gidx[...] = jnp.where(tv16 == 1, sv16, 0)
                pltpu.async_copy(t1h.at[gidx], rowsB, sem).wait()
                gidx[...] = jnp.where(tv16 == 2, sv16, 0)
                pltpu.async_copy(t2h.at[gidx], rowsC, sem).wait()
                for l in range(_L):
                    pos = kk * _L + l

                    @pl.when((pos >= lead) & (pos < lead + slen))
                    def _():
                        tb = tv16[l]
                        for q in range(h // _L):
                            qs = pl.ds(q * _L, _L)
                            pick = jnp.where(
                                tb == 0, rowsA[l, qs],
                                jnp.where(tb == 1, rowsB[l, qs],
                                          rowsC[l, qs]))
                            accrow[qs] = accrow[qs] + pick
                return cc

            lax.fori_loop(0, (lead + slen + _L - 1) // _L, ch_body, 0)
            for l in range(_L):
                for q in range(h // _L):
                    rowsA[l, pl.ds(q * _L, _L)] = accrow[pl.ds(q * _L, _L)]
            sdst[...] = jnp.full((_L,), 0, _i32) + srank
            pltpu.sync_copy(rowsA, outh.at[sdst])
            return c

        seg_trips = jnp.maximum(0, (n_seg - w + _NW - 1) // _NW)
        lax.fori_loop(0, seg_trips, seg_body, 0)

        pltpu.sync_copy(zinh, zrows)
        ntail_chunks = (t_total - n_uniq + _C - 1) // _C

        def z_body(i, c):
            base = n_uniq + (w + i * _NW) * _C
            for kk in range(_C // _L):
                v = base + kk * _L + iota16
                ztail[pl.ds(kk * _L, _L)] = jnp.where(v < t_total, v, n_uniq)
            pltpu.sync_copy(zrows, outh.at[ztail])
            return c

        z_trips = jnp.maximum(0, (ntail_chunks - w + _NW - 1) // _NW)
        lax.fori_loop(0, z_trips, z_body, 0)

    return k(t0, t1, t2, s0, d0, s1, d1, s2, d2,
             melem_src, melem_tab, segmeta, msmall, zin)


def kernel(diag_x, pair_x, node_h, edge_h, W_diag, W_pair, pair_index,
           edge_index):
    n = node_h.shape[0]
    p = pair_x.shape[0]
    e = edge_h.shape[0]
    h = node_h.shape[1]
    t_total = p + e + n

    pair_value = _matmul_call(pair_x, W_pair, 2000)
    node_value = _matmul_add_call(node_h, diag_x, W_diag, 2000)

    idt = pair_index.dtype
    pk = pair_index[0] * n + pair_index[1]
    ek = edge_index[0] * n + edge_index[1]
    nr = jnp.arange(n, dtype=idt)
    dk = nr * (n + 1)
    key = jnp.concatenate([pk, ek, dk])
    skey, perm = lax.sort(
        [key, jnp.arange(t_total, dtype=_i32)], num_keys=1)
    newf = jnp.concatenate([
        jnp.ones((1,), _i32), (skey[1:] != skey[:-1]).astype(_i32)])
    rank = jnp.cumsum(newf, dtype=_i32) - 1
    n_uniq = rank[-1] + 1

    idxa = jnp.arange(t_total, dtype=_i32)
    newf_next = jnp.concatenate([newf[1:], jnp.ones((1,), _i32)])
    single = (newf == 1) & (newf_next == 1)
    dstdir = jnp.where(single, rank, -1)

    m1 = (perm >= p) & (perm < p + e)
    m2 = perm >= p + e
    tabid = m1.astype(_i32) + 2 * m2.astype(_i32)
    src_local = perm - jnp.where(m1, p, 0) - jnp.where(m2, p + e, 0)

    m0 = ~(m1 | m2)
    c0 = jnp.cumsum(m0.astype(_i32))
    c1 = jnp.cumsum(m1.astype(_i32))
    c2 = jnp.cumsum(m2.astype(_i32))
    slot = (jnp.where(m0, c0 - 1, 0) + jnp.where(m1, p + c1 - 1, 0)
            + jnp.where(m2, p + e + c2 - 1, 0))
    sd = jnp.stack([src_local, dstdir], axis=1)
    sd_sorted = jnp.zeros((t_total, 2), _i32).at[slot].set(sd)
    src_sorted = sd_sorted[:, 0]
    dst_sorted = sd_sorted[:, 1]

    n2pad = -(-n // _C) * _C
    s0, d0 = src_sorted[:p], dst_sorted[:p]
    s1, d1 = src_sorted[p:p + e], dst_sorted[p:p + e]
    s2 = jnp.concatenate(
        [src_sorted[p + e:], jnp.zeros((n2pad - n,), _i32)])
    d2 = jnp.concatenate(
        [dst_sorted[p + e:], jnp.full((n2pad - n,), -1, _i32)])

    def _donor_fill(s_t, d_t):
        sz = s_t.shape[0]
        valid = d_t >= 0
        idx0 = jnp.argmax(valid).astype(_i32)
        s_first = s_t[idx0]
        d_first = d_t[idx0]
        s_f = jnp.where(valid, s_t, s_first)
        d_f = jnp.where(valid, d_t, d_first)
        cc = jnp.where(jnp.any(valid), _i32(sz // _C), _i32(0))
        return s_f, d_f, cc

    s0, d0, cc0 = _donor_fill(s0, d0)
    s1, d1, cc1 = _donor_fill(s1, d1)
    s2, d2, cc2 = _donor_fill(s2, d2)

    sm = (newf == 1) & ~single
    n_seg = jnp.sum(sm.astype(_i32))
    segid = jnp.cumsum(sm.astype(_i32)) - 1
    segcap = t_total // 2 + 2
    segslot = jnp.where(sm, jnp.clip(segid, 0, segcap - 1), segcap - 1)
    segvals = jnp.stack(
        [jnp.where(sm, idxa, 0), jnp.zeros((t_total,), _i32),
         jnp.where(sm, rank, -1)] +
        [jnp.zeros((t_total,), _i32)] * (_SMW - 3), axis=1)
    mslot = jnp.where(~single, jnp.clip(segid, 0, segcap - 1), segcap - 1)
    segmeta2 = jnp.zeros((segcap, _SMW), _i32).at[segslot].set(segvals)
    segmeta2 = segmeta2.at[mslot, 1].add(1)
    segmeta = jnp.concatenate(
        [segmeta2.reshape(-1), jnp.zeros((_L,), _i32)])
    melem_src = src_local
    melem_tab = tabid
    msmall = jnp.concatenate(
        [jnp.stack([n_seg, n_uniq, cc0, cc1, cc2]), jnp.zeros((11,), _i32)])

    zin = jnp.zeros((_C, h), _f32)

    return _coalesce_call(pair_value, edge_h, node_value,
                          s0, d0, s1, d1, s2, d2,
                          melem_src, melem_tab, segmeta, msmall, zin,
                          t_total, h)


# speedup vs baseline: 2.0266x; 2.0266x over previous
import functools

import jax
import jax.numpy as jnp
from jax import lax
from jax.experimental import pallas as pl
from jax.experimental.pallas import tpu as pltpu
from jax.experimental.pallas import tpu_sc as plsc

_f32 = jnp.float32
_i32 = jnp.int32

_NC = 2
_NS = 16
_NW = _NC * _NS
_L = 16

_C = 128
_SMW = 8


def _matmul_call(x, w, bp):
    m, k = x.shape
    _, h = w.shape

    def body(x_ref, w_ref, o_ref):
        o_ref[...] = lax.dot_general(
            x_ref[...], w_ref[...], (((1,), (0,)), ((), ())),
            preferred_element_type=_f32)

    return pl.pallas_call(
        body,
        grid=(m // bp,),
        in_specs=[
            pl.BlockSpec((bp, k), lambda i: (i, 0)),
            pl.BlockSpec((k, h), lambda i: (0, 0)),
        ],
        out_specs=pl.BlockSpec((bp, h), lambda i: (i, 0)),
        out_shape=jax.ShapeDtypeStruct((m, h), _f32),
    )(x, w)


def _matmul_add_call(base, x, w, bp):
    m, k = x.shape
    _, h = w.shape

    def body(b_ref, x_ref, w_ref, o_ref):
        o_ref[...] = b_ref[...] + lax.dot_general(
            x_ref[...], w_ref[...], (((1,), (0,)), ((), ())),
            preferred_element_type=_f32)

    return pl.pallas_call(
        body,
        grid=(m // bp,),
        in_specs=[
            pl.BlockSpec((bp, h), lambda i: (i, 0)),
            pl.BlockSpec((bp, k), lambda i: (i, 0)),
            pl.BlockSpec((k, h), lambda i: (0, 0)),
        ],
        out_specs=pl.BlockSpec((bp, h), lambda i: (i, 0)),
        out_shape=jax.ShapeDtypeStruct((m, h), _f32),
    )(base, x, w)


def _coalesce_call(t0, t1, t2, s0, d0, s1, d1, s2, d2,
                   melem_src, melem_tab, segmeta, msmall, zin,
                   t_total, h):
    sizes = (s0.shape[0], s1.shape[0], s2.shape[0])

    mesh = plsc.VectorSubcoreMesh(
        core_axis_name="c", subcore_axis_name="s",
        num_cores=_NC, num_subcores=_NS)

    @functools.partial(
        pl.kernel,
        out_type=jax.ShapeDtypeStruct((t_total, h), _f32),
        mesh=mesh,
        scratch_types=[
            pltpu.VMEM((_C,), _i32),
            pltpu.VMEM((_C,), _i32),
            pltpu.VMEM((_C, h), _f32),
            pltpu.VMEM((_C, h), _f32),
            pltpu.VMEM((_C,), _i32),
            pltpu.VMEM((_L,), _i32),
            pltpu.VMEM((_L,), _i32),
            pltpu.VMEM((_L,), _i32),
            pltpu.VMEM((_L,), _i32),
            pltpu.VMEM((_L, h), _f32),
            pltpu.VMEM((_L, h), _f32),
            pltpu.VMEM((_L, h), _f32),
            pltpu.VMEM((h,), _f32),
            pltpu.VMEM((_L,), _i32),
            pltpu.SemaphoreType.DMA,
        ],
    )
    def k(t0h, t1h, t2h, s0h, d0h, s1h, d1h, s2h, d2h,
          msrch, mtabh, segh, msmallh, zinh, outh,
          pidx, destv, rows, zrows, ztail, mb16, sidx, stab, gidx,
          rowsA, rowsB, rowsC, accrow, sdst, sem):
        cid = lax.axis_index("c")
        sid = lax.axis_index("s")
        w = cid * _NS + sid
        iota16 = lax.broadcasted_iota(_i32, (_L,), 0)

        pltpu.sync_copy(msmallh, mb16)
        mv = mb16[...]
        n_seg = mv[0]
        n_uniq = mv[1]

        for t, (tab, srch, dsth, sz) in enumerate(
                ((t0h, s0h, d0h, sizes[0]), (t1h, s1h, d1h, sizes[1]),
                 (t2h, s2h, d2h, sizes[2]))):
            nchunks = mv[2 + t]

            def a_body(i, c, tab=tab, srch=srch, dsth=dsth):
                g = w + i * _NW
                off = pl.multiple_of(g * _C, _C)
                pltpu.sync_copy(srch.at[pl.ds(off, _C)], pidx)
                pltpu.sync_copy(dsth.at[pl.ds(off, _C)], destv)
                pltpu.async_copy(tab.at[pidx], rows, sem).wait()
                pltpu.sync_copy(rows, outh.at[destv])
                return c

            trips = jnp.maximum(0, (nchunks - w + _NW - 1) // _NW)
            lax.fori_loop(0, trips, a_body, 0)


        def seg_body(i, c):
            m = w + i * _NW
            soff = pl.multiple_of(m * _SMW, _SMW)
            pltpu.sync_copy(segh.at[pl.ds(soff, _L)], mb16)
            sv = mb16[...]
            sstart = sv[0]
            slen = sv[1]
            srank = sv[2]
            a0 = (sstart // _L) * _L
            lead = sstart - a0
            for q in range(h // _L):
                accrow[pl.ds(q * _L, _L)] = jnp.zeros((_L,), _f32)

            def ch_body(kk, cc):
                eoff = pl.multiple_of(a0, _L) + kk * _L
                pltpu.sync_copy(msrch.at[pl.ds(eoff, _L)], sidx)
                pltpu.sync_copy(mtabh.at[pl.ds(eoff, _L)], stab)
                sv16 = sidx[...]
                tv16 = stab[...]
                gidx[...] = jnp.where(tv16 == 0, sv16, 0)
                pltpu.async_copy(t0h.at[gidx], rowsA, sem).wait()
                gidx[...] = jnp.where(tv16 == 1, sv16, 0)
                pltpu.async_copy(t1h.at[gidx], rowsB, sem).wait()
                gidx[...] = jnp.where(tv16 == 2, sv16, 0)
                pltpu.async_copy(t2h.at[gidx], rowsC, sem).wait()
                for l in range(_L):
                    pos = kk * _L + l

                    @pl.when((pos >= lead) & (pos < lead + slen))
                    def _():
                        tb = tv16[l]
                        for q in range(h // _L):
                            qs = pl.ds(q * _L, _L)
                            pick = jnp.where(
                                tb == 0, rowsA[l, qs],
                                jnp.where(tb == 1, rowsB[l, qs],
                                          rowsC[l, qs]))
                            accrow[qs] = accrow[qs] + pick
                return cc

            lax.fori_loop(0, (lead + slen + _L - 1) // _L, ch_body, 0)
            for l in range(_L):
                for q in range(h // _L):
                    rowsA[l, pl.ds(q * _L, _L)] = accrow[pl.ds(q * _L, _L)]
            sdst[...] = jnp.full((_L,), 0, _i32) + srank
            pltpu.sync_copy(rowsA, outh.at[sdst])
            return c

        seg_trips = jnp.maximum(0, (n_seg - w + _NW - 1) // _NW)
        lax.fori_loop(0, seg_trips, seg_body, 0)

        pltpu.sync_copy(zinh, zrows)
        ntail_chunks = (t_total - n_uniq + _C - 1) // _C

        def z_body(i, c):
            base = n_uniq + (w + i * _NW) * _C
            for kk in range(_C // _L):
                v = base + kk * _L + iota16
                ztail[pl.ds(kk * _L, _L)] = jnp.where(v < t_total, v, n_uniq)
            pltpu.sync_copy(zrows, outh.at[ztail])
            return c

        z_trips = jnp.maximum(0, (ntail_chunks - w + _NW - 1) // _NW)
        lax.fori_loop(0, z_trips, z_body, 0)

    return k(t0, t1, t2, s0, d0, s1, d1, s2, d2,
             melem_src, melem_tab, segmeta, msmall, zin)


def kernel(diag_x, pair_x, node_h, edge_h, W_diag, W_pair, pair_index,
           edge_index):
    n = node_h.shape[0]
    p = pair_x.shape[0]
    e = edge_h.shape[0]
    h = node_h.shape[1]
    t_total = p + e + n

    pair_value = _matmul_call(pair_x, W_pair, 2000)
    node_value = _matmul_add_call(node_h, diag_x, W_diag, 2000)

    idt = pair_index.dtype
    pk = pair_index[0] * n + pair_index[1]
    ek = edge_index[0] * n + edge_index[1]
    nr = jnp.arange(n, dtype=idt)
    dk = nr * (n + 1)
    key = jnp.concatenate([pk, ek, dk])
    skey, perm = lax.sort(
        [key, jnp.arange(t_total, dtype=_i32)], num_keys=1)
    newf = jnp.concatenate([
        jnp.ones((1,), _i32), (skey[1:] != skey[:-1]).astype(_i32)])
    rank = jnp.cumsum(newf, dtype=_i32) - 1
    n_uniq = rank[-1] + 1

    idxa = jnp.arange(t_total, dtype=_i32)
    newf_next = jnp.concatenate([newf[1:], jnp.ones((1,), _i32)])
    single = (newf == 1) & (newf_next == 1)
    dstdir = jnp.where(single, rank, -1)

    m1 = (perm >= p) & (perm < p + e)
    m2 = perm >= p + e
    tabid = m1.astype(_i32) + 2 * m2.astype(_i32)
    src_local = perm - jnp.where(m1, p, 0) - jnp.where(m2, p + e, 0)

    m0 = ~(m1 | m2)
    c0 = jnp.cumsum(m0.astype(_i32))
    c1 = jnp.cumsum(m1.astype(_i32))
    c2 = jnp.cumsum(m2.astype(_i32))
    slot = (jnp.where(m0, c0 - 1, 0) + jnp.where(m1, p + c1 - 1, 0)
            + jnp.where(m2, p + e + c2 - 1, 0))
    sd = jnp.stack([src_local, dstdir], axis=1)
    sd_sorted = jnp.zeros((t_total, 2), _i32).at[slot].add(sd)
    src_sorted = sd_sorted[:, 0]
    dst_sorted = sd_sorted[:, 1]

    n2pad = -(-n // _C) * _C
    s0, d0 = src_sorted[:p], dst_sorted[:p]
    s1, d1 = src_sorted[p:p + e], dst_sorted[p:p + e]
    s2 = jnp.concatenate(
        [src_sorted[p + e:], jnp.zeros((n2pad - n,), _i32)])
    d2 = jnp.concatenate(
        [dst_sorted[p + e:], jnp.full((n2pad - n,), -1, _i32)])

    def _donor_fill(s_t, d_t):
        sz = s_t.shape[0]
        valid = d_t >= 0
        idx0 = jnp.argmax(valid).astype(_i32)
        s_first = s_t[idx0]
        d_first = d_t[idx0]
        s_f = jnp.where(valid, s_t, s_first)
        d_f = jnp.where(valid, d_t, d_first)
        cc = jnp.where(jnp.any(valid), _i32(sz // _C), _i32(0))
        return s_f, d_f, cc

    s0, d0, cc0 = _donor_fill(s0, d0)
    s1, d1, cc1 = _donor_fill(s1, d1)
    s2, d2, cc2 = _donor_fill(s2, d2)

    sm = (newf == 1) & ~single
    n_seg = jnp.sum(sm.astype(_i32))
    segid = jnp.cumsum(sm.astype(_i32)) - 1
    segcap = t_total // 2 + 2
    segslot = jnp.where(sm, jnp.clip(segid, 0, segcap - 1), segcap - 1)
    segvals = jnp.stack(
        [jnp.where(sm, idxa, 0), jnp.zeros((t_total,), _i32),
         jnp.where(sm, rank, -1)] +
        [jnp.zeros((t_total,), _i32)] * (_SMW - 3), axis=1)
    mslot = jnp.where(~single, jnp.clip(segid, 0, segcap - 1), segcap - 1)
    segmeta2 = jnp.zeros((segcap, _SMW), _i32).at[segslot].add(segvals)
    segmeta2 = segmeta2.at[mslot, 1].add(1)
    segmeta = jnp.concatenate(
        [segmeta2.reshape(-1), jnp.zeros((_L,), _i32)])
    melem_src = src_local
    melem_tab = tabid
    msmall = jnp.concatenate(
        [jnp.stack([n_seg, n_uniq, cc0, cc1, cc2]), jnp.zeros((11,), _i32)])

    zin = jnp.zeros((_C, h), _f32)

    return _coalesce_call(pair_value, edge_h, node_value,
                          s0, d0, s1, d1, s2, d2,
                          melem_src, melem_tab, segmeta, msmall, zin,
                          t_total, h)


# speedup vs baseline: 4.6472x; 2.2931x over previous
import functools

import jax
import jax.numpy as jnp
from jax import lax
from jax.experimental import pallas as pl
from jax.experimental.pallas import tpu as pltpu
from jax.experimental.pallas import tpu_sc as plsc

_f32 = jnp.float32
_i32 = jnp.int32

_NC = 2
_NS = 16
_NW = _NC * _NS
_L = 16

_C = 128
_SMW = 8


def _matmul_call(x, w, bp):
    m, k = x.shape
    _, h = w.shape

    def body(x_ref, w_ref, o_ref):
        o_ref[...] = lax.dot_general(
            x_ref[...], w_ref[...], (((1,), (0,)), ((), ())),
            preferred_element_type=_f32)

    return pl.pallas_call(
        body,
        grid=(m // bp,),
        in_specs=[
            pl.BlockSpec((bp, k), lambda i: (i, 0)),
            pl.BlockSpec((k, h), lambda i: (0, 0)),
        ],
        out_specs=pl.BlockSpec((bp, h), lambda i: (i, 0)),
        out_shape=jax.ShapeDtypeStruct((m, h), _f32),
    )(x, w)


def _matmul_add_call(base, x, w, bp):
    m, k = x.shape
    _, h = w.shape

    def body(b_ref, x_ref, w_ref, o_ref):
        o_ref[...] = b_ref[...] + lax.dot_general(
            x_ref[...], w_ref[...], (((1,), (0,)), ((), ())),
            preferred_element_type=_f32)

    return pl.pallas_call(
        body,
        grid=(m // bp,),
        in_specs=[
            pl.BlockSpec((bp, h), lambda i: (i, 0)),
            pl.BlockSpec((bp, k), lambda i: (i, 0)),
            pl.BlockSpec((k, h), lambda i: (0, 0)),
        ],
        out_specs=pl.BlockSpec((bp, h), lambda i: (i, 0)),
        out_shape=jax.ShapeDtypeStruct((m, h), _f32),
    )(base, x, w)


def _coalesce_call(t0, t1, t2, s0, d0, s1, d1, s2, d2,
                   melem_src, melem_tab, segmeta, msmall, zin,
                   t_total, h):
    sizes = (s0.shape[0], s1.shape[0], s2.shape[0])

    mesh = plsc.VectorSubcoreMesh(
        core_axis_name="c", subcore_axis_name="s",
        num_cores=_NC, num_subcores=_NS)

    @functools.partial(
        pl.kernel,
        out_type=jax.ShapeDtypeStruct((t_total, h), _f32),
        mesh=mesh,
        scratch_types=[
            pltpu.VMEM((_C,), _i32),
            pltpu.VMEM((_C,), _i32),
            pltpu.VMEM((_C, h), _f32),
            pltpu.VMEM((_C, h), _f32),
            pltpu.VMEM((_C,), _i32),
            pltpu.VMEM((_L,), _i32),
            pltpu.VMEM((_L,), _i32),
            pltpu.VMEM((_L,), _i32),
            pltpu.VMEM((_L,), _i32),
            pltpu.VMEM((_L, h), _f32),
            pltpu.VMEM((_L, h), _f32),
            pltpu.VMEM((_L, h), _f32),
            pltpu.VMEM((h,), _f32),
            pltpu.VMEM((_L,), _i32),
            pltpu.SemaphoreType.DMA,
        ],
    )
    def k(t0h, t1h, t2h, s0h, d0h, s1h, d1h, s2h, d2h,
          msrch, mtabh, segh, msmallh, zinh, outh,
          pidx, destv, rows, zrows, ztail, mb16, sidx, stab, gidx,
          rowsA, rowsB, rowsC, accrow, sdst, sem):
        cid = lax.axis_index("c")
        sid = lax.axis_index("s")
        w = cid * _NS + sid
        iota16 = lax.broadcasted_iota(_i32, (_L,), 0)

        pltpu.sync_copy(msmallh, mb16)
        mv = mb16[...]
        n_seg = mv[0]
        n_uniq = mv[1]

        for t, (tab, srch, dsth, sz) in enumerate(
                ((t0h, s0h, d0h, sizes[0]), (t1h, s1h, d1h, sizes[1]),
                 (t2h, s2h, d2h, sizes[2]))):
            nchunks = mv[2 + t]

            def a_body(i, c, tab=tab, srch=srch, dsth=dsth):
                g = w + i * _NW
                off = pl.multiple_of(g * _C, _C)
                pltpu.sync_copy(srch.at[pl.ds(off, _C)], pidx)
                pltpu.sync_copy(dsth.at[pl.ds(off, _C)], destv)
                pltpu.async_copy(tab.at[pidx], rows, sem).wait()
                pltpu.sync_copy(rows, outh.at[destv])
                return c

            trips = jnp.maximum(0, (nchunks - w + _NW - 1) // _NW)
            lax.fori_loop(0, trips, a_body, 0)


        def seg_body(i, c):
            m = w + i * _NW
            soff = pl.multiple_of(m * _SMW, _SMW)
            pltpu.sync_copy(segh.at[pl.ds(soff, _L)], mb16)
            sv = mb16[...]
            sstart = sv[0]
            slen = sv[1]
            srank = sv[2]
            a0 = (sstart // _L) * _L
            lead = sstart - a0
            for q in range(h // _L):
                accrow[pl.ds(q * _L, _L)] = jnp.zeros((_L,), _f32)

            def ch_body(kk, cc):
                eoff = pl.multiple_of(a0, _L) + kk * _L
                pltpu.sync_copy(msrch.at[pl.ds(eoff, _L)], sidx)
                pltpu.sync_copy(mtabh.at[pl.ds(eoff, _L)], stab)
                sv16 = sidx[...]
                tv16 = stab[...]
                gidx[...] = jnp.where(tv16 == 0, sv16, 0)
                pltpu.async_copy(t0h.at[gidx], rowsA, sem).wait()
                gidx[...] = jnp.where(tv16 == 1, sv16, 0)
                pltpu.async_copy(t1h.at[gidx], rowsB, sem).wait()
                gidx[...] = jnp.where(tv16 == 2, sv16, 0)
                pltpu.async_copy(t2h.at[gidx], rowsC, sem).wait()
                for l in range(_L):
                    pos = kk * _L + l

                    @pl.when((pos >= lead) & (pos < lead + slen))
                    def _():
                        tb = tv16[l]
                        for q in range(h // _L):
                            qs = pl.ds(q * _L, _L)
                            pick = jnp.where(
                                tb == 0, rowsA[l, qs],
                                jnp.where(tb == 1, rowsB[l, qs],
                                          rowsC[l, qs]))
                            accrow[qs] = accrow[qs] + pick
                return cc

            lax.fori_loop(0, (lead + slen + _L - 1) // _L, ch_body, 0)
            for l in range(_L):
                for q in range(h // _L):
                    rowsA[l, pl.ds(q * _L, _L)] = accrow[pl.ds(q * _L, _L)]
            sdst[...] = jnp.full((_L,), 0, _i32) + srank
            pltpu.sync_copy(rowsA, outh.at[sdst])
            return c

        seg_trips = jnp.maximum(0, (n_seg - w + _NW - 1) // _NW)
        lax.fori_loop(0, seg_trips, seg_body, 0)

        pltpu.sync_copy(zinh, zrows)
        ntail_chunks = (t_total - n_uniq + _C - 1) // _C

        def z_body(i, c):
            base = n_uniq + (w + i * _NW) * _C
            for kk in range(_C // _L):
                v = base + kk * _L + iota16
                ztail[pl.ds(kk * _L, _L)] = jnp.where(v < t_total, v, n_uniq)
            pltpu.sync_copy(zrows, outh.at[ztail])
            return c

        z_trips = jnp.maximum(0, (ntail_chunks - w + _NW - 1) // _NW)
        lax.fori_loop(0, z_trips, z_body, 0)

    return k(t0, t1, t2, s0, d0, s1, d1, s2, d2,
             melem_src, melem_tab, segmeta, msmall, zin)


def kernel(diag_x, pair_x, node_h, edge_h, W_diag, W_pair, pair_index,
           edge_index):
    n = node_h.shape[0]
    p = pair_x.shape[0]
    e = edge_h.shape[0]
    h = node_h.shape[1]
    t_total = p + e + n

    pair_value = _matmul_call(pair_x, W_pair, 2000)
    node_value = _matmul_add_call(node_h, diag_x, W_diag, 2000)

    idt = pair_index.dtype
    pk = pair_index[0] * n + pair_index[1]
    ek = edge_index[0] * n + edge_index[1]
    nr = jnp.arange(n, dtype=idt)
    dk = nr * (n + 1)
    key = jnp.concatenate([pk, ek, dk])
    skey, perm = lax.sort(
        [key, jnp.arange(t_total, dtype=_i32)], num_keys=1)
    newf = jnp.concatenate([
        jnp.ones((1,), _i32), (skey[1:] != skey[:-1]).astype(_i32)])
    rank = jnp.cumsum(newf, dtype=_i32) - 1
    n_uniq = rank[-1] + 1

    idxa = jnp.arange(t_total, dtype=_i32)
    newf_next = jnp.concatenate([newf[1:], jnp.ones((1,), _i32)])
    single = (newf == 1) & (newf_next == 1)
    dstdir = jnp.where(single, rank, -1)

    m1 = (perm >= p) & (perm < p + e)
    m2 = perm >= p + e
    tabid = m1.astype(_i32) + 2 * m2.astype(_i32)
    src_local = perm - jnp.where(m1, p, 0) - jnp.where(m2, p + e, 0)

    dst_all = (jnp.zeros((t_total,), _i32).at[perm].add(
        jnp.where(single, rank + 1, 0)) - 1)

    n2pad = -(-n // _C) * _C
    d0 = dst_all[:p]
    d1 = dst_all[p:p + e]
    d2 = jnp.concatenate(
        [dst_all[p + e:], jnp.full((n2pad - n,), -1, _i32)])

    def _donor_fill(d_t):
        sz = d_t.shape[0]
        valid = d_t >= 0
        idx0 = jnp.argmax(valid).astype(_i32)
        d_first = d_t[idx0]
        ii = jnp.arange(sz, dtype=_i32)
        s_f = jnp.where(valid, ii, idx0)
        d_f = jnp.where(valid, d_t, d_first)
        cc = jnp.where(jnp.any(valid), _i32(sz // _C), _i32(0))
        return s_f, d_f, cc

    s0, d0, cc0 = _donor_fill(d0)
    s1, d1, cc1 = _donor_fill(d1)
    s2, d2, cc2 = _donor_fill(d2)

    sm = (newf == 1) & ~single
    n_seg = jnp.sum(sm.astype(_i32))
    segid = jnp.cumsum(sm.astype(_i32)) - 1
    segcap = t_total // 2 + 2
    segslot = jnp.where(sm, jnp.clip(segid, 0, segcap - 1), segcap - 1)
    mslot = jnp.where(~single, jnp.clip(segid, 0, segcap - 1), segcap - 1)
    segf = jnp.zeros((segcap * _SMW,), _i32)
    segf = segf.at[segslot * _SMW].add(jnp.where(sm, idxa, 0))
    segf = segf.at[mslot * _SMW + 1].add(1)
    segf = segf.at[segslot * _SMW + 2].add(jnp.where(sm, rank, -1))
    segmeta = jnp.concatenate([segf, jnp.zeros((_L,), _i32)])
    melem_src = src_local
    melem_tab = tabid
    msmall = jnp.concatenate(
        [jnp.stack([n_seg, n_uniq, cc0, cc1, cc2]), jnp.zeros((11,), _i32)])

    zin = jnp.zeros((_C, h), _f32)

    return _coalesce_call(pair_value, edge_h, node_value,
                          s0, d0, s1, d1, s2, d2,
                          melem_src, melem_tab, segmeta, msmall, zin,
                          t_total, h)
